# baseline (speedup 1.0000x reference)
import dataclasses
import numpy as np
import jax
from jax import lax
import jax.numpy as jnp
from jax.experimental import pallas as pl
from jax.experimental.pallas import tpu as pltpu
from jax.experimental.pallas import tpu_sc as plsc


def _anchor_table(box_size):
    ratios = np.asarray([0.5, 1.0, 2.0], dtype=np.float64)
    scales = np.asarray([1.0, 2.0 ** (1.0 / 3.0), 2.0 ** (2.0 / 3.0)],
                        dtype=np.float64)
    anchors = box_size * np.tile(scales, (2, len(ratios))).T
    areas = anchors[:, 0] * anchors[:, 1]
    anchors[:, 0] = np.sqrt(areas * np.repeat(ratios, len(scales)))
    anchors[:, 1] = anchors[:, 0] / np.repeat(ratios, len(scales))
    return anchors.astype(np.float32)


_LEVELS = (
    (64, 6, 8.0, 4, 16, 72, 0, _anchor_table(32)),
    (32, 5, 16.0, 8, 4, 72, 1152, _anchor_table(64)),
    (16, 4, 32.0, 16, 1, 72, 1440, _anchor_table(128)),
    (8, 3, 64.0, 8, 1, 18, 1512, _anchor_table(256)),
)

_ROWS = 1530
_N = 48960
_FR = 48
_CH = _FR * 32
_TFR = _ROWS - 31 * _FR
_TCH = _TFR * 32


def _select9(a, consts):
    out = jnp.float32(float(consts[8]))
    for k in range(7, -1, -1):
        out = jnp.where(a == k, jnp.float32(float(consts[k])), out)
    return out


def _tc_fill_body(buf_a, buf_x):
    for (W, log2w, s, hpp, reps, prows, roff, tab) in _LEVELS:
        r = jax.lax.broadcasted_iota(jnp.int32, (prows, 128), 0)
        c = jax.lax.broadcasted_iota(jnp.int32, (prows, 128), 1)
        f = r * 128 + c
        i = f >> 2
        j = f & 3
        q = ((i.astype(jnp.float32) + 0.5) * (1.0 / 9.0)).astype(jnp.int32)
        a = i - q * 9
        w = (q & (W - 1)).astype(jnp.float32)
        h = (q >> log2w).astype(jnp.float32)
        cx = (w + 0.5) * s
        cy = (h + 0.5) * s
        wa = _select9(a, tab[:, 0])
        ha = _select9(a, tab[:, 1])
        pat_a = jnp.where(j == 0, cx,
                jnp.where(j == 1, cy,
                jnp.where(j == 2, wa, ha)))
        pat_x = jnp.where(j == 0, cx - 0.5 * wa,
                jnp.where(j == 1, cy - 0.5 * ha,
                jnp.where(j == 2, cx + 0.5 * wa, cy + 0.5 * ha)))
        step = jnp.float32(hpp * s)
        msk_a = jnp.where(j == 1, step, jnp.float32(0.0))
        msk_x = jnp.where((j & 1) == 1, step, jnp.float32(0.0))
        cur_a, cur_x = pat_a, pat_x
        for g in range(reps):
            buf_a[roff + g * prows: roff + (g + 1) * prows, :] = cur_a
            buf_x[roff + g * prows: roff + (g + 1) * prows, :] = cur_x
            if g + 1 < reps:
                cur_a = cur_a + msk_a
                cur_x = cur_x + msk_x


def _tc_fill():
    return pl.pallas_call(
        _tc_fill_body,
        out_shape=[
            jax.ShapeDtypeStruct((_ROWS, 128), jnp.float32),
            jax.ShapeDtypeStruct((_ROWS, 128), jnp.float32),
        ],
    )()


def _sc_distribute(flat_a, flat_x):
    mesh = plsc.VectorSubcoreMesh(core_axis_name="c", subcore_axis_name="s")
    cp = pltpu.CompilerParams()
    if "needs_layout_passes" in pltpu.CompilerParams.__dataclass_fields__:
        cp = dataclasses.replace(cp, needs_layout_passes=False)
    cp = dataclasses.replace(cp, use_tc_tiling_on_sc=False)

    @pl.kernel(
        compiler_params=cp,
        out_type=[
            jax.ShapeDtypeStruct((_N, 4), jnp.float32),
            jax.ShapeDtypeStruct((_N, 4), jnp.float32),
        ],
        mesh=mesh,
        scratch_types=[
            pltpu.VMEM((_FR, 128), jnp.float32),
            pltpu.VMEM((_CH, 4), jnp.float32),
            pltpu.VMEM((_CH, 4), jnp.float32),
        ],
    )
    def sc_kernel(flat_a_hbm, flat_x_hbm, out_a_hbm, out_x_hbm,
                  b_flat, b4_a, b4_x):
        wid = lax.axis_index("s") * 2 + lax.axis_index("c")
        lanes = lax.iota(jnp.int32, 16)
        idx_c = lanes & 3
        idx_r0 = lanes >> 2

        def move(flat_hbm, b4, out_hbm, flat_off, box_off, fr, ch):
            pltpu.sync_copy(flat_hbm.at[pl.ds(flat_off, fr), :],
                            b_flat.at[pl.ds(0, fr), :])

            @pl.loop(0, ch * 4 // 16)
            def _relayout(v, b4=b4):
                val = b_flat[v >> 3, pl.ds((v & 7) * 16, 16)]
                plsc.store_scatter(b4, [idx_r0 + 4 * v, idx_c], val)

            pltpu.sync_copy(b4.at[pl.ds(0, ch), :],
                            out_hbm.at[pl.ds(box_off, ch), :])

        pairs = ((flat_a_hbm, b4_a, out_a_hbm), (flat_x_hbm, b4_x, out_x_hbm))

        @pl.when(wid < 31)
        def _main():
            for flat_hbm, b4, out_hbm in pairs:
                move(flat_hbm, b4, out_hbm, wid * _FR, wid * _CH, _FR, _CH)

        @pl.when(wid == 31)
        def _tail():
            for flat_hbm, b4, out_hbm in pairs:
                move(flat_hbm, b4, out_hbm, 31 * _FR, 31 * _CH, _TFR, _TCH)

    out_a, out_x = sc_kernel(flat_a, flat_x)
    return out_a, out_x


def kernel(feat0, feat1, feat2, feat3):
    del feat0, feat1, feat2, feat3
    flat_a, flat_x = _tc_fill()
    return _sc_distribute(flat_a, flat_x)


# speedup vs baseline: 1.9049x; 1.9049x over previous
import numpy as np
import jax
import jax.numpy as jnp
from jax.experimental import pallas as pl
from jax.experimental.pallas import tpu as pltpu


def _anchor_table(box_size):
    ratios = np.asarray([0.5, 1.0, 2.0], dtype=np.float64)
    scales = np.asarray([1.0, 2.0 ** (1.0 / 3.0), 2.0 ** (2.0 / 3.0)],
                        dtype=np.float64)
    anchors = box_size * np.tile(scales, (2, len(ratios))).T
    areas = anchors[:, 0] * anchors[:, 1]
    anchors[:, 0] = np.sqrt(areas * np.repeat(ratios, len(scales)))
    anchors[:, 1] = anchors[:, 0] / np.repeat(ratios, len(scales))
    return anchors.astype(np.float32)


_C = 576
_N = 48960
_NDMA = 6
_CHUNK = _N // _NDMA

_LEVELS = (
    (64, 6, 8.0, 1, 64, 0, _anchor_table(32)),
    (32, 5, 16.0, 2, 16, 36864, _anchor_table(64)),
    (16, 4, 32.0, 4, 4, 46080, _anchor_table(128)),
    (8, 3, 64.0, 8, 1, 48384, _anchor_table(256)),
)


def _select9(a, consts):
    out = jnp.float32(float(consts[8]))
    for k in range(7, -1, -1):
        out = jnp.where(a == k, jnp.float32(float(consts[k])), out)
    return out


def _fill(buf_a, buf_x):
    i = jax.lax.broadcasted_iota(jnp.int32, (_C, 4), 0)
    j = jax.lax.broadcasted_iota(jnp.int32, (_C, 4), 1)
    for (W, log2w, s, hpc, reps, roff, tab) in _LEVELS:
        q = ((i.astype(jnp.float32) + 0.5) * (1.0 / 9.0)).astype(jnp.int32)
        a = i - q * 9
        w = (q & (W - 1)).astype(jnp.float32)
        h = (q >> log2w).astype(jnp.float32)
        cx = (w + 0.5) * s
        cy = (h + 0.5) * s
        wa = _select9(a, tab[:, 0])
        ha = _select9(a, tab[:, 1])
        cur_a = jnp.where(j == 0, cx,
                jnp.where(j == 1, cy,
                jnp.where(j == 2, wa, ha)))
        cur_x = jnp.where(j == 0, cx - 0.5 * wa,
                jnp.where(j == 1, cy - 0.5 * ha,
                jnp.where(j == 2, cx + 0.5 * wa, cy + 0.5 * ha)))
        step = jnp.float32(hpc * s)
        msk_a = jnp.where(j == 1, step, jnp.float32(0.0))
        msk_x = jnp.where((j & 1) == 1, step, jnp.float32(0.0))
        for g in range(reps):
            o = roff + g * _C
            buf_a[pl.ds(o, _C), :] = cur_a
            buf_x[pl.ds(o, _C), :] = cur_x
            if g + 1 < reps:
                cur_a = cur_a + msk_a
                cur_x = cur_x + msk_x


def _body(out_a_hbm, out_x_hbm, buf_a, buf_x, sem):
    _fill(buf_a, buf_x)
    cps = []
    for k in range(_NDMA):
        rows = pl.ds(k * _CHUNK, _CHUNK)
        cp_a = pltpu.async_copy(
            buf_a.at[rows, :], out_a_hbm.at[rows, :], sem.at[2 * k],
            priority=(2 * k) % 2)
        cp_x = pltpu.async_copy(
            buf_x.at[rows, :], out_x_hbm.at[rows, :], sem.at[2 * k + 1],
            priority=(2 * k + 1) % 2)
        cps += [cp_a, cp_x]
    for cp in cps:
        cp.wait()


def kernel(feat0, feat1, feat2, feat3):
    del feat0, feat1, feat2, feat3
    return pl.pallas_call(
        _body,
        out_shape=[
            jax.ShapeDtypeStruct((_N, 4), jnp.float32),
            jax.ShapeDtypeStruct((_N, 4), jnp.float32),
        ],
        out_specs=[
            pl.BlockSpec(memory_space=pl.ANY),
            pl.BlockSpec(memory_space=pl.ANY),
        ],
        scratch_shapes=[
            pltpu.VMEM((_N, 4), jnp.float32),
            pltpu.VMEM((_N, 4), jnp.float32),
            pltpu.SemaphoreType.DMA((2 * _NDMA,)),
        ],
    )()
